# baseline (speedup 1.0000x reference)
import functools

import numpy as np
import jax
import jax.numpy as jnp
from jax import lax
from jax.experimental import pallas as pl
from jax.experimental.pallas import tpu as pltpu
from jax.experimental.pallas import tpu_sc as plsc

NX = 1000
NUM_GRIDS = NX * NX
DIM = 64
_DX32 = np.float32(1.0) / np.float32(1000)
_CX = np.float32(1.0 / float(_DX32))
_CY = np.float32(1000.0 / float(_DX32))

_info = plsc.get_sparse_core_info()
NC, NS, L = _info.num_cores, _info.num_subcores, _info.num_lanes
NW = NC * NS

BP = 128
NB = 4
LAG = 3


@functools.partial(jax.jit, static_argnums=(0, 1))
def _grid_ebd_sc(Bt, H, t4, W):
    NBJ = Bt // BP
    NU = H * NBJ
    UW = NU // NW
    mesh = plsc.VectorSubcoreMesh(core_axis_name="c", subcore_axis_name="s")

    @functools.partial(
        pl.kernel,
        out_type=jax.ShapeDtypeStruct((H, DIM // 8, NBJ, 8 * BP), jnp.float32),
        mesh=mesh,
        compiler_params=pltpu.CompilerParams(
            needs_layout_passes=False, use_tc_tiling_on_sc=False
        ),
        scratch_types=(
            [pltpu.VMEM((2, BP), jnp.float32) for _ in range(NB)]
            + [pltpu.VMEM((BP,), jnp.int32) for _ in range(NB)]
            + [pltpu.VMEM((BP, DIM), jnp.float32) for _ in range(NB)]
            + [pltpu.VMEM((DIM // 8, 8 * BP), jnp.float32) for _ in range(NB)]
            + [pltpu.SemaphoreType.DMA for _ in range(4 * NB)]
        ),
    )
    def k(t_hbm, w_hbm, out_hbm, *bufs):
        tv = bufs[0:NB]
        iv = bufs[NB:2 * NB]
        rv = bufs[2 * NB:3 * NB]
        ov = bufs[3 * NB:4 * NB]
        st = bufs[4 * NB:5 * NB]
        sg = bufs[5 * NB:6 * NB]
        so = bufs[6 * NB:7 * NB]
        wid = lax.axis_index("s") * NC + lax.axis_index("c")
        u0 = wid * UW
        lane = lax.iota(jnp.int32, L)
        arow = [(qq * L + lane) // 8 for qq in range(4)]
        rbase = [((qq * L + lane) % 8) * BP for qq in range(4)]

        def hj(u):
            ug = u0 + u
            return ug // NBJ, ug % NBJ

        def tload(u, q):
            h, jb = hj(u)
            pltpu.async_copy(t_hbm.at[h, jb], tv[q], st[q])

        def compute_idx(q):
            for s in range(BP // L):
                x = tv[q][0, pl.ds(s * L, L)]
                y = tv[q][1, pl.ds(s * L, L)]
                g = (x * _CX).astype(jnp.int32) + (y * _CY).astype(jnp.int32)
                g = jnp.where((g > NUM_GRIDS) | (g < 0), NUM_GRIDS, g)
                iv[q][pl.ds(s * L, L)] = g

        def step_fire(u, q, do_tload=True):
            pltpu.make_async_copy(t_hbm.at[0, 0], tv[q], st[q]).wait()
            compute_idx(q)
            if do_tload:
                tload(u + 1, (q + 1) % NB)
            pltpu.async_copy(w_hbm.at[iv[q]], rv[q], sg[q])

        def transpose(q):
            def body_pt(m, carry):
                for uu in range(4):
                    pt = m * 4 + uu
                    for qq in range(4):
                        v = rv[q][pt, pl.ds(qq * L, L)]
                        plsc.store_scatter(
                            ov[q], [arow[qq], rbase[qq] + pt], v
                        )
                return carry

            lax.fori_loop(0, BP // 4, body_pt, 0)

        def step_finish(v, q, do_odrain=True):
            pltpu.make_async_copy(w_hbm.at[pl.ds(0, BP)], rv[q], sg[q]).wait()
            if do_odrain:
                pltpu.make_async_copy(
                    ov[q], out_hbm.at[0, :, 0, :], so[q]
                ).wait()
            transpose(q)
            h, jb = hj(v)
            pltpu.async_copy(ov[q], out_hbm.at[h, :, jb, :], so[q])

        tload(0, 0)
        for u in range(LAG):
            step_fire(u, u % NB)
        for u in range(LAG, 2 * NB):
            step_fire(u, u % NB)
            step_finish(u - LAG, (u - LAG) % NB, do_odrain=(u - LAG >= NB))

        def body(m, carry):
            for j in range(NB):
                u = 2 * NB + NB * m + j
                step_fire(u, j)
                step_finish(u - LAG, (j + 1) % NB)
            return carry

        NQ = (UW - 2 * NB - NB) // NB
        lax.fori_loop(0, NQ, body, 0)

        for u in range(2 * NB + NB * NQ, UW):
            step_fire(u, u % NB, do_tload=(u + 1 < UW))
            step_finish(u - LAG, (u - LAG) % NB)
        for v in range(UW - LAG, UW):
            step_finish(v, v % NB)
        for q in range(NB):
            pltpu.make_async_copy(ov[q], out_hbm.at[0, :, 0, :], so[q]).wait()

    return k(t4, W)


def kernel(T, W):
    Bt, H, _ = T.shape
    t4 = T.transpose(1, 0, 2).reshape(H, Bt // BP, BP, 2).transpose(0, 1, 3, 2)
    out4 = _grid_ebd_sc(Bt, H, t4, W)
    out5 = out4.reshape(H, DIM // 8, Bt // BP, 8, BP)
    return out5.transpose(2, 4, 0, 1, 3).reshape(Bt, H, DIM)
